# baseline (speedup 1.0000x reference)
import functools

import jax
import jax.numpy as jnp
import numpy as np
from jax import lax
from jax.experimental import pallas as pl
from jax.experimental.pallas import tpu as pltpu
from jax.experimental.pallas import tpu_sc as plsc

_A = 9
_STRIDE = 16
_PRE = 6000
_POST = 300
_THRESH = 0.7
_NEG = -1e30
_VALID_TH = -5e29

_B = 4
_N = 36864
_M = 6144
_NSUB = 8
_CHUNK = _N // _NSUB
_NJ = _CHUNK // 128
_NW = _B * _NSUB
_NR = _B * _N // 128
_NC = _N // 128
_DUMP = _B * _M
_OUTN = _DUMP + 64


def _host_anchors(feature_h, feature_w):
    base_size = 16.0
    ratios = np.array([0.5, 1.0, 2.0])
    scales = np.array([8.0, 16.0, 32.0])
    x_ctr = (base_size - 1.0) / 2.0
    y_ctr = (base_size - 1.0) / 2.0
    size = base_size * base_size
    rows = []
    for r in ratios:
        ws = np.round(np.sqrt(size / r))
        hs = np.round(ws * r)
        for s in scales:
            w = ws * s
            h = hs * s
            rows.append([x_ctr - 0.5 * (w - 1.0), y_ctr - 0.5 * (h - 1.0),
                         x_ctr + 0.5 * (w - 1.0), y_ctr + 0.5 * (h - 1.0)])
    base = np.asarray(rows, dtype=np.float32)
    shift_x = np.arange(feature_w, dtype=np.float32) * _STRIDE
    shift_y = np.arange(feature_h, dtype=np.float32) * _STRIDE
    sx, sy = np.meshgrid(shift_x, shift_y)
    shifts = np.stack([sx.ravel(), sy.ravel(), sx.ravel(), sy.ravel()],
                      axis=1).astype(np.float32)
    return (shifts[:, None, :] + base[None, :, :]).reshape(-1, 4)


def _decode_body(sc_ref, dx_ref, dy_ref, dw_ref, dh_ref,
                 ax1_ref, ay1_ref, ax2_ref, ay2_ref, h_ref, w_ref,
                 x1_o, y1_o, x2_o, y2_o, msc_o, key_s):
    B, N = sc_ref.shape
    iota = lax.broadcasted_iota(jnp.int32, (B, N), 1)

    ax1 = ax1_ref[...]
    ay1 = ay1_ref[...]
    ax2 = ax2_ref[...]
    ay2 = ay2_ref[...]
    aw = ax2 - ax1 + 1.0
    ah = ay2 - ay1 + 1.0
    acx = ax1 + 0.5 * aw
    acy = ay1 + 0.5 * ah
    pcx = dx_ref[...] * aw + acx
    pcy = dy_ref[...] * ah + acy
    pw = jnp.exp(dw_ref[...]) * aw
    ph = jnp.exp(dh_ref[...]) * ah
    px1 = pcx - 0.5 * pw
    py1 = pcy - 0.5 * ph
    px2 = pcx + 0.5 * pw
    py2 = pcy + 0.5 * ph
    hh = h_ref[...]
    ww = w_ref[...]
    x1_o[...] = jnp.clip(px1, 0.0, ww - 1.0)
    y1_o[...] = jnp.clip(py1, 0.0, hh - 1.0)
    x2_o[...] = jnp.clip(px2, 0.0, ww - 1.0)
    y2_o[...] = jnp.clip(py2, 0.0, hh - 1.0)

    sc = sc_ref[...]
    bits = lax.bitcast_convert_type(sc, jnp.int32)
    key_s[...] = jnp.where(bits < 0, bits ^ jnp.int32(0x7FFFFFFF), bits)
    min32 = jnp.int32(-2147483648)

    def bs_val(i, u):
        b = 31 - i
        cand_u = u | jnp.left_shift(jnp.int32(1), b)
        thr = min32 + cand_u
        cnt = jnp.sum((key_s[...] >= thr).astype(jnp.int32),
                      axis=1, keepdims=True)
        return jnp.where(cnt >= _PRE, cand_u, u)

    u = lax.fori_loop(0, 32, bs_val, jnp.zeros((B, 1), jnp.int32))
    t6 = min32 + u

    keyv = key_s[...]
    cnt_gt = jnp.sum((keyv > t6).astype(jnp.int32), axis=1, keepdims=True)
    quota = _PRE - cnt_gt

    def bs_idx(i, lohi):
        lo, hi = lohi
        mid = (lo + hi) >> 1
        g = jnp.sum(((key_s[...] == t6) & (iota <= mid)).astype(jnp.int32),
                    axis=1, keepdims=True)
        ok = g >= quota
        return jnp.where(ok, lo, mid + 1), jnp.where(ok, mid, hi)

    lo0 = jnp.zeros((B, 1), jnp.int32)
    hi0 = jnp.full((B, 1), N - 1, jnp.int32)
    _, bound = lax.fori_loop(0, 17, bs_idx, (lo0, hi0))

    sel = (keyv > t6) | ((keyv == t6) & (iota <= bound))
    msc_o[...] = jnp.where(sel, sc, jnp.float32(_NEG))


def _lane_prefix_body(msc_ref, pfx_o, tot_o):
    sel = jnp.where(msc_ref[...] > jnp.float32(_VALID_TH),
                    jnp.float32(1.0), jnp.float32(0.0))
    r = lax.broadcasted_iota(jnp.int32, (128, 128), 0)
    c = lax.broadcasted_iota(jnp.int32, (128, 128), 1)
    tri = jnp.where(r < c, jnp.float32(1.0), jnp.float32(0.0))
    pfx_o[...] = lax.dot_general(sel, tri, (((1,), (0,)), ((), ())),
                                 preferred_element_type=jnp.float32)
    tot_o[...] = jnp.sum(sel, axis=1, keepdims=True)


def _chunk_prefix_body(tot_ref, cp_o):
    r = lax.broadcasted_iota(jnp.int32, (_NC, _NC), 0)
    c = lax.broadcasted_iota(jnp.int32, (_NC, _NC), 1)
    tri = jnp.where(r < c, jnp.float32(1.0), jnp.float32(0.0))
    cp_o[...] = lax.dot_general(tot_ref[...], tri, (((1,), (0,)), ((), ())),
                                preferred_element_type=jnp.float32)


def _dest_body(msc_ref, pfx_ref, cp_ref, dest_o):
    sel = msc_ref[...] > jnp.float32(_VALID_TH)
    pos = (cp_ref[...] + pfx_ref[...]).astype(jnp.int32)
    rowbase = (lax.broadcasted_iota(jnp.int32, (_NR, 1), 0) // _NC) * _M
    lane = lax.broadcasted_iota(jnp.int32, (_NR, 128), 1)
    dump = jnp.int32(_DUMP) + (lane & 63)
    dest_o[...] = jnp.where(sel, rowbase + pos, dump)


def _scatter_body(x1h, y1h, x2h, y2h, msch, desth,
                  ox1, oy1, ox2, oy2, omsc,
                  x1v, y1v, x2v, y2v, mscv, destv, sem):
    c = lax.axis_index("c")
    s = lax.axis_index("s")
    w = (c * 2 + s // _NSUB) * _NSUB + s % _NSUB

    srcs = (x1v, y1v, x2v, y2v, mscv)
    hbms = (x1h, y1h, x2h, y2h, msch)
    outs = (ox1, oy1, ox2, oy2, omsc)
    for hv, vv in zip(hbms, srcs):
        pltpu.sync_copy(hv.at[w], vv)
    pltpu.sync_copy(desth.at[w], destv)

    for st in range(5):
        def sbody(j, _, _src=srcs[st], _out=outs[st]):
            pltpu.make_async_copy(
                _src.at[j], _out.at[destv.at[j]], sem).start()
            return 0

        lax.fori_loop(0, _NJ, sbody, 0)
    for st in range(5):
        pltpu.make_async_copy(hbms[st].at[w], srcs[st], sem).wait()


def _nms_body(x1_ref, y1_ref, x2_ref, y2_ref, mscin_ref,
              out_ref, ar_s, msc_s):
    B, M = mscin_ref.shape
    iota = lax.broadcasted_iota(jnp.int32, (B, M), 1)
    x1 = x1_ref[...]
    y1 = y1_ref[...]
    x2 = x2_ref[...]
    y2 = y2_ref[...]
    ar_s[...] = (x2 - x1 + 1.0) * (y2 - y1 + 1.0)
    msc_s[...] = jnp.where(iota < _PRE, mscin_ref[...], jnp.float32(_NEG))
    bcol = lax.broadcasted_iota(jnp.int32, (B, 1), 0).astype(jnp.float32)

    def nms(i, _):
        msc = msc_s[...]
        m = jnp.max(msc, axis=1, keepdims=True)
        idx = jnp.min(jnp.where(msc == m, iota, jnp.int32(M)),
                      axis=1, keepdims=True)
        selm = iota == idx
        x1 = x1_ref[...]
        y1 = y1_ref[...]
        x2 = x2_ref[...]
        y2 = y2_ref[...]
        fill = jnp.float32(-3.0e38)
        cx1 = jnp.max(jnp.where(selm, x1, fill), axis=1, keepdims=True)
        cy1 = jnp.max(jnp.where(selm, y1, fill), axis=1, keepdims=True)
        cx2 = jnp.max(jnp.where(selm, x2, fill), axis=1, keepdims=True)
        cy2 = jnp.max(jnp.where(selm, y2, fill), axis=1, keepdims=True)
        carea = (cx2 - cx1 + 1.0) * (cy2 - cy1 + 1.0)
        valid = (m > jnp.float32(_NEG * 0.5)).astype(jnp.float32)
        xx1 = jnp.maximum(cx1, x1)
        yy1 = jnp.maximum(cy1, y1)
        xx2 = jnp.minimum(cx2, x2)
        yy2 = jnp.minimum(cy2, y2)
        iw = jnp.maximum(xx2 - xx1 + 1.0, 0.0)
        ih = jnp.maximum(yy2 - yy1 + 1.0, 0.0)
        inter = iw * ih
        iou = inter / (carea + ar_s[...] - inter)
        msc_s[...] = jnp.where((iou > _THRESH) | selm, jnp.float32(_NEG), msc)
        row = jnp.concatenate(
            [bcol, cx1 * valid, cy1 * valid, cx2 * valid, cy2 * valid], axis=1)
        out_ref[i, :, :] = row
        return 0

    lax.fori_loop(0, _POST, nms, 0)


def kernel(scores_raw, bbox_deltas, im_info):
    B = scores_raw.shape[0]
    H, W = scores_raw.shape[2], scores_raw.shape[3]
    N = H * W * _A
    f32 = jnp.float32

    sc = jnp.transpose(scores_raw[:, _A:], (0, 2, 3, 1)).reshape(B, N)
    d = jnp.transpose(bbox_deltas, (0, 2, 3, 1)).reshape(B, N, 4)
    dx, dy, dw, dh = d[..., 0], d[..., 1], d[..., 2], d[..., 3]

    anch = _host_anchors(H, W)
    ax1 = jnp.asarray(anch[:, 0]).reshape(1, N)
    ay1 = jnp.asarray(anch[:, 1]).reshape(1, N)
    ax2 = jnp.asarray(anch[:, 2]).reshape(1, N)
    ay2 = jnp.asarray(anch[:, 3]).reshape(1, N)
    hcol = im_info[:, 0:1].astype(f32)
    wcol = im_info[:, 1:2].astype(f32)

    x1f, y1f, x2f, y2f, mscf = pl.pallas_call(
        _decode_body,
        out_shape=[jax.ShapeDtypeStruct((B, N), f32)] * 5,
        in_specs=[pl.BlockSpec(memory_space=pltpu.VMEM)] * 11,
        out_specs=[pl.BlockSpec(memory_space=pltpu.VMEM)] * 5,
        scratch_shapes=[pltpu.VMEM((B, N), jnp.int32)],
    )(sc, dx, dy, dw, dh, ax1, ay1, ax2, ay2, hcol, wcol)

    msc2 = mscf.reshape(_NR, 128)
    pfx, tot = pl.pallas_call(
        _lane_prefix_body,
        out_shape=[jax.ShapeDtypeStruct((_NR, 128), f32),
                   jax.ShapeDtypeStruct((_NR, 1), f32)],
        in_specs=[pl.BlockSpec(memory_space=pltpu.VMEM)],
        out_specs=[pl.BlockSpec(memory_space=pltpu.VMEM)] * 2,
    )(msc2)

    cp = pl.pallas_call(
        _chunk_prefix_body,
        out_shape=jax.ShapeDtypeStruct((B, _NC), f32),
        in_specs=[pl.BlockSpec(memory_space=pltpu.VMEM)],
        out_specs=pl.BlockSpec(memory_space=pltpu.VMEM),
    )(tot.reshape(B, _NC))

    dest = pl.pallas_call(
        _dest_body,
        out_shape=jax.ShapeDtypeStruct((_NR, 128), jnp.int32),
        in_specs=[pl.BlockSpec(memory_space=pltpu.VMEM)] * 3,
        out_specs=pl.BlockSpec(memory_space=pltpu.VMEM),
    )(msc2, pfx, cp.reshape(_NR, 1))

    mesh = plsc.VectorSubcoreMesh(core_axis_name="c", subcore_axis_name="s")
    scatter = functools.partial(
        pl.kernel,
        out_type=[jax.ShapeDtypeStruct((_OUTN,), f32)] * 5,
        mesh=mesh,
        scratch_types=[
            pltpu.VMEM((_NJ, 128), f32),
            pltpu.VMEM((_NJ, 128), f32),
            pltpu.VMEM((_NJ, 128), f32),
            pltpu.VMEM((_NJ, 128), f32),
            pltpu.VMEM((_NJ, 128), f32),
            pltpu.VMEM((_NJ, 128), jnp.int32),
            pltpu.SemaphoreType.DMA,
        ],
    )(_scatter_body)

    def slab(a):
        return a.reshape(_NW, _NJ, 128)

    cx1, cy1, cx2, cy2, cmsc = scatter(
        slab(x1f), slab(y1f), slab(x2f), slab(y2f), slab(mscf), slab(dest))

    def rows(a):
        return a[: B * _M].reshape(B, _M)

    out = pl.pallas_call(
        _nms_body,
        out_shape=jax.ShapeDtypeStruct((_POST, B, 5), f32),
        in_specs=[pl.BlockSpec(memory_space=pltpu.VMEM)] * 5,
        out_specs=pl.BlockSpec(memory_space=pltpu.VMEM),
        scratch_shapes=[
            pltpu.VMEM((B, _M), f32),
            pltpu.VMEM((B, _M), f32),
        ],
    )(rows(cx1), rows(cy1), rows(cx2), rows(cy2), rows(cmsc))
    return jnp.transpose(out, (1, 0, 2))


# speedup vs baseline: 67.1490x; 67.1490x over previous
import functools

import jax
import jax.numpy as jnp
import numpy as np
from jax.experimental import pallas as pl
from jax.experimental.pallas import tpu as pltpu

_A = 9
_STRIDE = 16
_PRE = 6000
_POST = 300
_THRESH = 0.7
_NEG = -1e30


def _host_anchors(feature_h, feature_w):
    base_size = 16.0
    ratios = np.array([0.5, 1.0, 2.0])
    scales = np.array([8.0, 16.0, 32.0])
    x_ctr = (base_size - 1.0) / 2.0
    y_ctr = (base_size - 1.0) / 2.0
    size = base_size * base_size
    rows = []
    for r in ratios:
        ws = np.round(np.sqrt(size / r))
        hs = np.round(ws * r)
        for s in scales:
            w = ws * s
            h = hs * s
            rows.append([x_ctr - 0.5 * (w - 1.0), y_ctr - 0.5 * (h - 1.0),
                         x_ctr + 0.5 * (w - 1.0), y_ctr + 0.5 * (h - 1.0)])
    base = np.asarray(rows, dtype=np.float32)
    shift_x = np.arange(feature_w, dtype=np.float32) * _STRIDE
    shift_y = np.arange(feature_h, dtype=np.float32) * _STRIDE
    sx, sy = np.meshgrid(shift_x, shift_y)
    shifts = np.stack([sx.ravel(), sy.ravel(), sx.ravel(), sy.ravel()], axis=1).astype(np.float32)
    return (shifts[:, None, :] + base[None, :, :]).reshape(-1, 4)


def _body(sc_ref, dx_ref, dy_ref, dw_ref, dh_ref,
          ax1_ref, ay1_ref, ax2_ref, ay2_ref, h_ref, w_ref,
          out_ref,
          x1_s, y1_s, x2_s, y2_s, ar_s, msc_s, key_s):
    B, N = sc_ref.shape
    iota = jax.lax.broadcasted_iota(jnp.int32, (B, N), 1)

    ax1 = ax1_ref[...]
    ay1 = ay1_ref[...]
    ax2 = ax2_ref[...]
    ay2 = ay2_ref[...]
    aw = ax2 - ax1 + 1.0
    ah = ay2 - ay1 + 1.0
    acx = ax1 + 0.5 * aw
    acy = ay1 + 0.5 * ah
    pcx = dx_ref[...] * aw + acx
    pcy = dy_ref[...] * ah + acy
    pw = jnp.exp(dw_ref[...]) * aw
    ph = jnp.exp(dh_ref[...]) * ah
    px1 = pcx - 0.5 * pw
    py1 = pcy - 0.5 * ph
    px2 = pcx + 0.5 * pw
    py2 = pcy + 0.5 * ph
    hh = h_ref[...]
    ww = w_ref[...]
    x1 = jnp.clip(px1, 0.0, ww - 1.0)
    y1 = jnp.clip(py1, 0.0, hh - 1.0)
    x2 = jnp.clip(px2, 0.0, ww - 1.0)
    y2 = jnp.clip(py2, 0.0, hh - 1.0)
    x1_s[...] = x1
    y1_s[...] = y1
    x2_s[...] = x2
    y2_s[...] = y2
    ar_s[...] = (x2 - x1 + 1.0) * (y2 - y1 + 1.0)

    sc = sc_ref[...]
    bits = jax.lax.bitcast_convert_type(sc, jnp.int32)
    key = jnp.where(bits < 0, bits ^ jnp.int32(0x7FFFFFFF), bits)
    key_s[...] = key
    min32 = jnp.int32(-2147483648)

    def bs_val(i, u):
        b = 31 - i
        cand_u = u | jnp.left_shift(jnp.int32(1), b)
        thr = min32 + cand_u
        cnt = jnp.sum((key_s[...] >= thr).astype(jnp.int32), axis=1, keepdims=True)
        return jnp.where(cnt >= _PRE, cand_u, u)

    u = jax.lax.fori_loop(0, 32, bs_val, jnp.zeros((B, 1), jnp.int32))
    t6 = min32 + u

    keyv = key_s[...]
    cnt_gt = jnp.sum((keyv > t6).astype(jnp.int32), axis=1, keepdims=True)
    quota = _PRE - cnt_gt

    def bs_idx(i, lohi):
        lo, hi = lohi
        mid = (lo + hi) >> 1
        g = jnp.sum(((key_s[...] == t6) & (iota <= mid)).astype(jnp.int32),
                    axis=1, keepdims=True)
        ok = g >= quota
        return jnp.where(ok, lo, mid + 1), jnp.where(ok, mid, hi)

    lo0 = jnp.zeros((B, 1), jnp.int32)
    hi0 = jnp.full((B, 1), N - 1, jnp.int32)
    _, bound = jax.lax.fori_loop(0, 17, bs_idx, (lo0, hi0))

    sel = (keyv > t6) | ((keyv == t6) & (iota <= bound))
    msc_s[...] = jnp.where(sel, sc, jnp.float32(_NEG))

    bcol = jax.lax.broadcasted_iota(jnp.int32, (B, 1), 0).astype(jnp.float32)

    lane = jax.lax.broadcasted_iota(jnp.int32, (1, 128), 1)

    def nms(i, _):
        msc = msc_s[...]
        m = jnp.max(msc, axis=1, keepdims=True)
        idx = jnp.min(jnp.where(msc == m, iota, jnp.int32(N)),
                      axis=1, keepdims=True)
        selm = iota == idx
        x1 = x1_s[...]
        y1 = y1_s[...]
        x2 = x2_s[...]
        y2 = y2_s[...]
        fill = jnp.float32(-3.0e38)
        c1r, c2r, c3r, c4r = [], [], [], []
        for b in range(B):
            ib = jnp.sum(jax.lax.slice(idx, (b, 0), (b + 1, 1)))
            cb = pl.multiple_of((ib // 128) * 128, 128)
            lb = ib - cb
            selc = lane == lb
            for lst, ref in ((c1r, x1_s), (c2r, y1_s), (c3r, x2_s), (c4r, y2_s)):
                ch = ref[pl.ds(b, 1), pl.ds(cb, 128)]
                lst.append(jnp.max(jnp.where(selc, ch, fill),
                                   axis=1, keepdims=True))
        cx1 = jnp.concatenate(c1r, axis=0)
        cy1 = jnp.concatenate(c2r, axis=0)
        cx2 = jnp.concatenate(c3r, axis=0)
        cy2 = jnp.concatenate(c4r, axis=0)
        carea = (cx2 - cx1 + 1.0) * (cy2 - cy1 + 1.0)
        valid = (m > jnp.float32(_NEG * 0.5)).astype(jnp.float32)
        xx1 = jnp.maximum(cx1, x1)
        yy1 = jnp.maximum(cy1, y1)
        xx2 = jnp.minimum(cx2, x2)
        yy2 = jnp.minimum(cy2, y2)
        iw = jnp.maximum(xx2 - xx1 + 1.0, 0.0)
        ih = jnp.maximum(yy2 - yy1 + 1.0, 0.0)
        inter = iw * ih
        iou = inter / (carea + ar_s[...] - inter)
        msc_s[...] = jnp.where((iou > _THRESH) | selm, jnp.float32(_NEG), msc)
        row = jnp.concatenate(
            [bcol, cx1 * valid, cy1 * valid, cx2 * valid, cy2 * valid], axis=1)
        out_ref[i, :, :] = row
        return 0

    jax.lax.fori_loop(0, _POST, nms, 0)


@functools.partial(jax.jit, static_argnames=())
def kernel(scores_raw, bbox_deltas, im_info):
    B = scores_raw.shape[0]
    H, W = scores_raw.shape[2], scores_raw.shape[3]
    N = H * W * _A
    f32 = jnp.float32

    sc = jnp.transpose(scores_raw[:, _A:], (0, 2, 3, 1)).reshape(B, N)
    d = jnp.transpose(bbox_deltas, (0, 2, 3, 1)).reshape(B, N, 4)
    dx, dy, dw, dh = d[..., 0], d[..., 1], d[..., 2], d[..., 3]

    anch = _host_anchors(H, W)
    ax1 = jnp.asarray(anch[:, 0]).reshape(1, N)
    ay1 = jnp.asarray(anch[:, 1]).reshape(1, N)
    ax2 = jnp.asarray(anch[:, 2]).reshape(1, N)
    ay2 = jnp.asarray(anch[:, 3]).reshape(1, N)
    hcol = im_info[:, 0:1].astype(f32)
    wcol = im_info[:, 1:2].astype(f32)

    out = pl.pallas_call(
        _body,
        out_shape=jax.ShapeDtypeStruct((_POST, B, 5), f32),
        in_specs=[pl.BlockSpec(memory_space=pltpu.VMEM)] * 11,
        out_specs=pl.BlockSpec(memory_space=pltpu.VMEM),
        scratch_shapes=[
            pltpu.VMEM((B, N), f32),
            pltpu.VMEM((B, N), f32),
            pltpu.VMEM((B, N), f32),
            pltpu.VMEM((B, N), f32),
            pltpu.VMEM((B, N), f32),
            pltpu.VMEM((B, N), f32),
            pltpu.VMEM((B, N), jnp.int32),
        ],
    )(sc, dx, dy, dw, dh, ax1, ay1, ax2, ay2, hcol, wcol)
    return jnp.transpose(out, (1, 0, 2))


# speedup vs baseline: 72.9442x; 1.0863x over previous
import functools

import jax
import jax.numpy as jnp
import numpy as np
from jax.experimental import pallas as pl
from jax.experimental.pallas import tpu as pltpu

_A = 9
_STRIDE = 16
_PRE = 6000
_POST = 300
_THRESH = 0.7
_NEG = -1e30


def _host_anchors(feature_h, feature_w):
    base_size = 16.0
    ratios = np.array([0.5, 1.0, 2.0])
    scales = np.array([8.0, 16.0, 32.0])
    x_ctr = (base_size - 1.0) / 2.0
    y_ctr = (base_size - 1.0) / 2.0
    size = base_size * base_size
    rows = []
    for r in ratios:
        ws = np.round(np.sqrt(size / r))
        hs = np.round(ws * r)
        for s in scales:
            w = ws * s
            h = hs * s
            rows.append([x_ctr - 0.5 * (w - 1.0), y_ctr - 0.5 * (h - 1.0),
                         x_ctr + 0.5 * (w - 1.0), y_ctr + 0.5 * (h - 1.0)])
    base = np.asarray(rows, dtype=np.float32)
    shift_x = np.arange(feature_w, dtype=np.float32) * _STRIDE
    shift_y = np.arange(feature_h, dtype=np.float32) * _STRIDE
    sx, sy = np.meshgrid(shift_x, shift_y)
    shifts = np.stack([sx.ravel(), sy.ravel(), sx.ravel(), sy.ravel()], axis=1).astype(np.float32)
    return (shifts[:, None, :] + base[None, :, :]).reshape(-1, 4)


def _body(sc_ref, dx_ref, dy_ref, dw_ref, dh_ref,
          ax1_ref, ay1_ref, ax2_ref, ay2_ref, h_ref, w_ref,
          out_ref,
          p1_s, p2_s, ar_s, msc_s, key_s):
    B, N = sc_ref.shape
    iota = jax.lax.broadcasted_iota(jnp.int32, (B, N), 1)

    ax1 = ax1_ref[...]
    ay1 = ay1_ref[...]
    ax2 = ax2_ref[...]
    ay2 = ay2_ref[...]
    aw = ax2 - ax1 + 1.0
    ah = ay2 - ay1 + 1.0
    acx = ax1 + 0.5 * aw
    acy = ay1 + 0.5 * ah
    pcx = dx_ref[...] * aw + acx
    pcy = dy_ref[...] * ah + acy
    pw = jnp.exp(dw_ref[...]) * aw
    ph = jnp.exp(dh_ref[...]) * ah
    px1 = pcx - 0.5 * pw
    py1 = pcy - 0.5 * ph
    px2 = pcx + 0.5 * pw
    py2 = pcy + 0.5 * ph
    hh = h_ref[...]
    ww = w_ref[...]
    x1 = jnp.clip(px1, 0.0, ww - 1.0)
    y1 = jnp.clip(py1, 0.0, hh - 1.0)
    x2 = jnp.clip(px2, 0.0, ww - 1.0)
    y2 = jnp.clip(py2, 0.0, hh - 1.0)
    p1_s[...] = jnp.concatenate([x1, y1], axis=0)
    p2_s[...] = jnp.concatenate([x2, y2], axis=0)
    ar_s[...] = (x2 - x1 + 1.0) * (y2 - y1 + 1.0)

    sc = sc_ref[...]
    bits = jax.lax.bitcast_convert_type(sc, jnp.int32)
    key = jnp.where(bits < 0, bits ^ jnp.int32(0x7FFFFFFF), bits)
    key_s[...] = key
    min32 = jnp.int32(-2147483648)

    def bs_val(i, u):
        b = 31 - i
        cand_u = u | jnp.left_shift(jnp.int32(1), b)
        thr = min32 + cand_u
        cnt = jnp.sum((key_s[...] >= thr).astype(jnp.int32), axis=1, keepdims=True)
        return jnp.where(cnt >= _PRE, cand_u, u)

    u = jax.lax.fori_loop(0, 32, bs_val, jnp.zeros((B, 1), jnp.int32))
    t6 = min32 + u

    keyv = key_s[...]
    cnt_gt = jnp.sum((keyv > t6).astype(jnp.int32), axis=1, keepdims=True)
    quota = _PRE - cnt_gt

    def bs_idx(i, lohi):
        lo, hi = lohi
        mid = (lo + hi) >> 1
        g = jnp.sum(((key_s[...] == t6) & (iota <= mid)).astype(jnp.int32),
                    axis=1, keepdims=True)
        ok = g >= quota
        return jnp.where(ok, lo, mid + 1), jnp.where(ok, mid, hi)

    lo0 = jnp.zeros((B, 1), jnp.int32)
    hi0 = jnp.full((B, 1), N - 1, jnp.int32)
    _, bound = jax.lax.fori_loop(0, 17, bs_idx, (lo0, hi0))

    sel = (keyv > t6) | ((keyv == t6) & (iota <= bound))
    msc_s[...] = jnp.where(sel, sc, jnp.float32(_NEG))

    bcol = jax.lax.broadcasted_iota(jnp.int32, (B, 1), 0).astype(jnp.float32)

    lane = jax.lax.broadcasted_iota(jnp.int32, (1, 128), 1)

    def nms(i, _):
        msc = msc_s[...]
        m = jnp.max(msc, axis=1, keepdims=True)
        idx = jnp.min(jnp.where(msc == m, iota, jnp.int32(N)),
                      axis=1, keepdims=True)
        selm = iota == idx
        fill = jnp.float32(-3.0e38)
        c1r, c2r = [], []
        for b in range(B):
            ib = jnp.sum(jax.lax.slice(idx, (b, 0), (b + 1, 1)))
            cb = pl.multiple_of((ib // 128) * 128, 128)
            lb = ib - cb
            selc = lane == lb
            for lst, ref in ((c1r, p1_s), (c2r, p2_s)):
                chx = ref[pl.ds(b, 1), pl.ds(cb, 128)]
                chy = ref[pl.ds(b + B, 1), pl.ds(cb, 128)]
                lst.append((jnp.max(jnp.where(selc, chx, fill),
                                    axis=1, keepdims=True),
                            jnp.max(jnp.where(selc, chy, fill),
                                    axis=1, keepdims=True)))
        cx1 = jnp.concatenate([t[0] for t in c1r], axis=0)
        cy1 = jnp.concatenate([t[1] for t in c1r], axis=0)
        cx2 = jnp.concatenate([t[0] for t in c2r], axis=0)
        cy2 = jnp.concatenate([t[1] for t in c2r], axis=0)
        carea = (cx2 - cx1 + 1.0) * (cy2 - cy1 + 1.0)
        valid = (m > jnp.float32(_NEG * 0.5)).astype(jnp.float32)
        cxy1 = jnp.concatenate([cx1, cy1], axis=0)
        cxy2 = jnp.concatenate([cx2, cy2], axis=0)
        xy1 = jnp.maximum(cxy1, p1_s[...])
        xy2 = jnp.minimum(cxy2, p2_s[...])
        d = jnp.maximum(xy2 - xy1 + 1.0, 0.0)
        iw = jax.lax.slice(d, (0, 0), (B, N))
        ih = jax.lax.slice(d, (B, 0), (2 * B, N))
        inter = iw * ih
        iou = inter / (carea + ar_s[...] - inter)
        msc_s[...] = jnp.where((iou > _THRESH) | selm, jnp.float32(_NEG), msc)
        row = jnp.concatenate(
            [bcol, cx1 * valid, cy1 * valid, cx2 * valid, cy2 * valid], axis=1)
        out_ref[i, :, :] = row
        return 0

    jax.lax.fori_loop(0, _POST, nms, 0)


@functools.partial(jax.jit, static_argnames=())
def kernel(scores_raw, bbox_deltas, im_info):
    B = scores_raw.shape[0]
    H, W = scores_raw.shape[2], scores_raw.shape[3]
    N = H * W * _A
    f32 = jnp.float32

    sc = jnp.transpose(scores_raw[:, _A:], (0, 2, 3, 1)).reshape(B, N)
    d = jnp.transpose(bbox_deltas, (0, 2, 3, 1)).reshape(B, N, 4)
    dx, dy, dw, dh = d[..., 0], d[..., 1], d[..., 2], d[..., 3]

    anch = _host_anchors(H, W)
    ax1 = jnp.asarray(anch[:, 0]).reshape(1, N)
    ay1 = jnp.asarray(anch[:, 1]).reshape(1, N)
    ax2 = jnp.asarray(anch[:, 2]).reshape(1, N)
    ay2 = jnp.asarray(anch[:, 3]).reshape(1, N)
    hcol = im_info[:, 0:1].astype(f32)
    wcol = im_info[:, 1:2].astype(f32)

    out = pl.pallas_call(
        _body,
        out_shape=jax.ShapeDtypeStruct((_POST, B, 5), f32),
        in_specs=[pl.BlockSpec(memory_space=pltpu.VMEM)] * 11,
        out_specs=pl.BlockSpec(memory_space=pltpu.VMEM),
        scratch_shapes=[
            pltpu.VMEM((2 * B, N), f32),
            pltpu.VMEM((2 * B, N), f32),
            pltpu.VMEM((B, N), f32),
            pltpu.VMEM((B, N), f32),
            pltpu.VMEM((B, N), jnp.int32),
        ],
    )(sc, dx, dy, dw, dh, ax1, ay1, ax2, ay2, hcol, wcol)
    return jnp.transpose(out, (1, 0, 2))


# speedup vs baseline: 90.2871x; 1.2378x over previous
import functools

import jax
import jax.numpy as jnp
import numpy as np
from jax import lax
from jax.experimental import pallas as pl
from jax.experimental.pallas import tpu as pltpu

_A = 9
_STRIDE = 16
_PRE = 6000
_POST = 300
_THRESH = 0.7
_NEG = -1e30


def _host_anchors(feature_h, feature_w):
    base_size = 16.0
    ratios = np.array([0.5, 1.0, 2.0])
    scales = np.array([8.0, 16.0, 32.0])
    x_ctr = (base_size - 1.0) / 2.0
    y_ctr = (base_size - 1.0) / 2.0
    size = base_size * base_size
    rows = []
    for r in ratios:
        ws = np.round(np.sqrt(size / r))
        hs = np.round(ws * r)
        for s in scales:
            w = ws * s
            h = hs * s
            rows.append([x_ctr - 0.5 * (w - 1.0), y_ctr - 0.5 * (h - 1.0),
                         x_ctr + 0.5 * (w - 1.0), y_ctr + 0.5 * (h - 1.0)])
    base = np.asarray(rows, dtype=np.float32)
    shift_x = np.arange(feature_w, dtype=np.float32) * _STRIDE
    shift_y = np.arange(feature_h, dtype=np.float32) * _STRIDE
    sx, sy = np.meshgrid(shift_x, shift_y)
    shifts = np.stack([sx.ravel(), sy.ravel(), sx.ravel(), sy.ravel()], axis=1).astype(np.float32)
    return (shifts[:, None, :] + base[None, :, :]).reshape(-1, 4)


def _body(sc_ref, dx_ref, dy_ref, dw_ref, dh_ref,
          ax1_ref, ay1_ref, ax2_ref, ay2_ref, h_ref, w_ref,
          out_ref,
          x1_s, y1_s, x2_s, y2_s, ar_s, msc_s, key_s):
    B8, H2 = sc_ref.shape
    B = B8 // 2
    N = 2 * H2
    subl = lax.broadcasted_iota(jnp.int32, (B8, 1), 0)
    iota = (lax.broadcasted_iota(jnp.int32, (B8, H2), 1)
            + jnp.where(subl >= B, jnp.int32(H2), 0))

    def comb_min(v8):
        return jnp.minimum(lax.slice(v8, (0, 0), (B, 1)),
                           lax.slice(v8, (B, 0), (B8, 1)))

    def comb_max(v8):
        return jnp.maximum(lax.slice(v8, (0, 0), (B, 1)),
                           lax.slice(v8, (B, 0), (B8, 1)))

    def comb_sum(v8):
        return (lax.slice(v8, (0, 0), (B, 1))
                + lax.slice(v8, (B, 0), (B8, 1)))

    def up(v4):
        return jnp.concatenate([v4, v4], axis=0)

    ax1 = ax1_ref[...]
    ay1 = ay1_ref[...]
    ax2 = ax2_ref[...]
    ay2 = ay2_ref[...]
    aw = ax2 - ax1 + 1.0
    ah = ay2 - ay1 + 1.0
    acx = ax1 + 0.5 * aw
    acy = ay1 + 0.5 * ah
    pcx = dx_ref[...] * aw + acx
    pcy = dy_ref[...] * ah + acy
    pw = jnp.exp(dw_ref[...]) * aw
    ph = jnp.exp(dh_ref[...]) * ah
    px1 = pcx - 0.5 * pw
    py1 = pcy - 0.5 * ph
    px2 = pcx + 0.5 * pw
    py2 = pcy + 0.5 * ph
    hh = h_ref[...]
    ww = w_ref[...]
    x1 = jnp.clip(px1, 0.0, ww - 1.0)
    y1 = jnp.clip(py1, 0.0, hh - 1.0)
    x2 = jnp.clip(px2, 0.0, ww - 1.0)
    y2 = jnp.clip(py2, 0.0, hh - 1.0)
    x1_s[...] = x1
    y1_s[...] = y1
    x2_s[...] = x2
    y2_s[...] = y2
    ar_s[...] = (x2 - x1 + 1.0) * (y2 - y1 + 1.0)

    sc = sc_ref[...]
    bits = lax.bitcast_convert_type(sc, jnp.int32)
    key = jnp.where(bits < 0, bits ^ jnp.int32(0x7FFFFFFF), bits)
    key_s[...] = key
    min32 = jnp.int32(-2147483648)

    def bs_val(i, u):
        b = 31 - i
        cand_u = u | jnp.left_shift(jnp.int32(1), b)
        thr8 = up(min32 + cand_u)
        cnt = comb_sum(jnp.sum((key_s[...] >= thr8).astype(jnp.int32),
                               axis=1, keepdims=True))
        return jnp.where(cnt >= _PRE, cand_u, u)

    u = lax.fori_loop(0, 32, bs_val, jnp.zeros((B, 1), jnp.int32))
    t6 = min32 + u
    t68 = up(t6)

    keyv = key_s[...]
    cnt_gt = comb_sum(jnp.sum((keyv > t68).astype(jnp.int32),
                              axis=1, keepdims=True))
    quota = _PRE - cnt_gt

    def bs_idx(i, lohi):
        lo, hi = lohi
        mid = (lo + hi) >> 1
        mid8 = up(mid)
        g = comb_sum(jnp.sum(((key_s[...] == t68) & (iota <= mid8))
                             .astype(jnp.int32), axis=1, keepdims=True))
        ok = g >= quota
        return jnp.where(ok, lo, mid + 1), jnp.where(ok, mid, hi)

    lo0 = jnp.zeros((B, 1), jnp.int32)
    hi0 = jnp.full((B, 1), N - 1, jnp.int32)
    _, bound = lax.fori_loop(0, 17, bs_idx, (lo0, hi0))

    sel = (keyv > t68) | ((keyv == t68) & (iota <= up(bound)))
    msc_s[...] = jnp.where(sel, sc, jnp.float32(_NEG))

    bcol = lax.broadcasted_iota(jnp.int32, (B, 1), 0).astype(jnp.float32)
    lane = lax.broadcasted_iota(jnp.int32, (1, 128), 1)

    def nms(i, _):
        msc = msc_s[...]
        m = comb_max(jnp.max(msc, axis=1, keepdims=True))
        m8 = up(m)
        idx = comb_min(jnp.min(jnp.where(msc == m8, iota, jnp.int32(N)),
                               axis=1, keepdims=True))
        idx8 = up(idx)
        selm = iota == idx8
        fill = jnp.float32(-3.0e38)
        rows = [[], [], [], []]
        for b in range(B):
            ib = jnp.sum(lax.slice(idx, (b, 0), (b + 1, 1)))
            half = ib // H2
            ibl = ib - half * H2
            cb = pl.multiple_of((ibl // 128) * 128, 128)
            lb = ibl - cb
            selc = lane == lb
            hf = half.astype(jnp.float32)
            for t, ref in enumerate((x1_s, y1_s, x2_s, y2_s)):
                ch0 = ref[pl.ds(b, 1), pl.ds(cb, 128)]
                ch1 = ref[pl.ds(b + B, 1), pl.ds(cb, 128)]
                ch = ch0 * (1.0 - hf) + ch1 * hf
                rows[t].append(jnp.max(jnp.where(selc, ch, fill),
                                       axis=1, keepdims=True))
        cx1 = jnp.concatenate(rows[0], axis=0)
        cy1 = jnp.concatenate(rows[1], axis=0)
        cx2 = jnp.concatenate(rows[2], axis=0)
        cy2 = jnp.concatenate(rows[3], axis=0)
        carea = (cx2 - cx1 + 1.0) * (cy2 - cy1 + 1.0)
        valid = (m > jnp.float32(_NEG * 0.5)).astype(jnp.float32)
        xx1 = jnp.maximum(up(cx1), x1_s[...])
        yy1 = jnp.maximum(up(cy1), y1_s[...])
        xx2 = jnp.minimum(up(cx2), x2_s[...])
        yy2 = jnp.minimum(up(cy2), y2_s[...])
        iw = jnp.maximum(xx2 - xx1 + 1.0, 0.0)
        ih = jnp.maximum(yy2 - yy1 + 1.0, 0.0)
        inter = iw * ih
        iou = inter / (up(carea) + ar_s[...] - inter)
        msc_s[...] = jnp.where((iou > _THRESH) | selm, jnp.float32(_NEG), msc)
        row = jnp.concatenate(
            [bcol, cx1 * valid, cy1 * valid, cx2 * valid, cy2 * valid], axis=1)
        out_ref[i, :, :] = row
        return 0

    lax.fori_loop(0, _POST, nms, 0)


@functools.partial(jax.jit, static_argnames=())
def kernel(scores_raw, bbox_deltas, im_info):
    B = scores_raw.shape[0]
    H, W = scores_raw.shape[2], scores_raw.shape[3]
    N = H * W * _A
    H2 = N // 2
    f32 = jnp.float32

    def split(a):
        return jnp.concatenate([a[:, :H2], a[:, H2:]], axis=0)

    sc = split(jnp.transpose(scores_raw[:, _A:], (0, 2, 3, 1)).reshape(B, N))
    d = jnp.transpose(bbox_deltas, (0, 2, 3, 1)).reshape(B, N, 4)
    dx, dy, dw, dh = (split(d[..., 0]), split(d[..., 1]),
                      split(d[..., 2]), split(d[..., 3]))

    anch = _host_anchors(H, W)

    def asplit(col):
        a2 = jnp.asarray(col).reshape(2, H2)
        return jnp.repeat(a2, B, axis=0)

    ax1 = asplit(anch[:, 0])
    ay1 = asplit(anch[:, 1])
    ax2 = asplit(anch[:, 2])
    ay2 = asplit(anch[:, 3])
    hcol = jnp.tile(im_info[:, 0:1].astype(f32), (2, 1))
    wcol = jnp.tile(im_info[:, 1:2].astype(f32), (2, 1))

    out = pl.pallas_call(
        _body,
        out_shape=jax.ShapeDtypeStruct((_POST, B, 5), f32),
        in_specs=[pl.BlockSpec(memory_space=pltpu.VMEM)] * 11,
        out_specs=pl.BlockSpec(memory_space=pltpu.VMEM),
        scratch_shapes=[
            pltpu.VMEM((2 * B, H2), f32),
            pltpu.VMEM((2 * B, H2), f32),
            pltpu.VMEM((2 * B, H2), f32),
            pltpu.VMEM((2 * B, H2), f32),
            pltpu.VMEM((2 * B, H2), f32),
            pltpu.VMEM((2 * B, H2), f32),
            pltpu.VMEM((2 * B, H2), jnp.int32),
        ],
    )(sc, dx, dy, dw, dh, ax1, ay1, ax2, ay2, hcol, wcol)
    return jnp.transpose(out, (1, 0, 2))
